# baseline (speedup 1.0000x reference)
import jax
import jax.numpy as jnp
from jax.experimental import pallas as pl
from jax.experimental.pallas import tpu as pltpu

_ROWS = 16384
_COLS = 1000
_FLAT = _ROWS * _COLS
_LANES = 128
_R2 = _FLAT // _LANES
_GRID = 16
_BLK = _R2 // _GRID


def _body(p_ref, t_ref, out_ref, acc_ref):
    i = pl.program_id(0)

    @pl.when(i == 0)
    def _init():
        acc_ref[0] = 0.0
        acc_ref[1] = 0.0

    x = p_ref[...]
    m = t_ref[...]
    logp = jnp.where(m, jnp.log(x), 0.0)
    acc_ref[0] += jnp.sum(logp)
    acc_ref[1] += jnp.sum(m.astype(jnp.float32))

    @pl.when(i == _GRID - 1)
    def _fin():
        out_ref[0] = -(acc_ref[0] / acc_ref[1])


def kernel(y_pred, y_true):
    p = y_pred.reshape(_R2, _LANES)
    t = y_true.reshape(_R2, _LANES)
    out = pl.pallas_call(
        _body,
        grid=(_GRID,),
        in_specs=[
            pl.BlockSpec((_BLK, _LANES), lambda i: (i, 0)),
            pl.BlockSpec((_BLK, _LANES), lambda i: (i, 0)),
        ],
        out_specs=pl.BlockSpec(memory_space=pltpu.SMEM),
        out_shape=jax.ShapeDtypeStruct((1,), jnp.float32),
        scratch_shapes=[pltpu.SMEM((2,), jnp.float32)],
    )(p, t)
    return out[0]


# speedup vs baseline: 1.0118x; 1.0118x over previous
import jax
import jax.numpy as jnp
from jax.experimental import pallas as pl
from jax.experimental.pallas import tpu as pltpu

_ROWS = 16384
_COLS = 1000
_FLAT = _ROWS * _COLS
_LANES = 128
_G = 8
_R2 = _FLAT // (_LANES * _G)
_GRID = 16
_BLK = _R2 // _GRID


def _body(p_ref, t_ref, out_ref, acc_ref):
    i = pl.program_id(0)

    @pl.when(i == 0)
    def _init():
        acc_ref[0] = 0.0
        acc_ref[1] = 0.0

    x = p_ref[...]
    m = t_ref[...]
    xs = jnp.where(m, x, 1.0)
    prod = xs[0]
    for g in range(1, _G):
        prod = prod * xs[g]
    acc_ref[0] += jnp.sum(jnp.log(prod))
    acc_ref[1] += jnp.sum(m.astype(jnp.float32))

    @pl.when(i == _GRID - 1)
    def _fin():
        out_ref[0] = -(acc_ref[0] / acc_ref[1])


def kernel(y_pred, y_true):
    p = y_pred.reshape(_G, _R2, _LANES)
    t = y_true.reshape(_G, _R2, _LANES)
    out = pl.pallas_call(
        _body,
        grid=(_GRID,),
        in_specs=[
            pl.BlockSpec((_G, _BLK, _LANES), lambda i: (0, i, 0)),
            pl.BlockSpec((_G, _BLK, _LANES), lambda i: (0, i, 0)),
        ],
        out_specs=pl.BlockSpec(memory_space=pltpu.SMEM),
        out_shape=jax.ShapeDtypeStruct((1,), jnp.float32),
        scratch_shapes=[pltpu.SMEM((2,), jnp.float32)],
    )(p, t)
    return out[0]


# speedup vs baseline: 1.7479x; 1.7275x over previous
import jax
import jax.numpy as jnp
from jax.experimental import pallas as pl
from jax.experimental.pallas import tpu as pltpu

_ROWS = 16384
_COLS = 1000
_GRID = 16
_BLK = _ROWS // _GRID
_G = 8
_CH = _BLK // _G


def _body(p_ref, t_ref, out_ref, acc_ref):
    i = pl.program_id(0)

    @pl.when(i == 0)
    def _init():
        acc_ref[0] = 0.0
        acc_ref[1] = 0.0

    prod = None
    cnt = None
    for g in range(_G):
        x = p_ref[pl.ds(g * _CH, _CH), :]
        m = t_ref[pl.ds(g * _CH, _CH), :]
        xs = jnp.where(m, x, 1.0)
        mf = m.astype(jnp.float32)
        prod = xs if prod is None else prod * xs
        cnt = mf if cnt is None else cnt + mf
    acc_ref[0] += jnp.sum(jnp.log(prod))
    acc_ref[1] += jnp.sum(cnt)

    @pl.when(i == _GRID - 1)
    def _fin():
        out_ref[0] = -(acc_ref[0] / acc_ref[1])


def kernel(y_pred, y_true):
    out = pl.pallas_call(
        _body,
        grid=(_GRID,),
        in_specs=[
            pl.BlockSpec((_BLK, _COLS), lambda i: (i, 0)),
            pl.BlockSpec((_BLK, _COLS), lambda i: (i, 0)),
        ],
        out_specs=pl.BlockSpec(memory_space=pltpu.SMEM),
        out_shape=jax.ShapeDtypeStruct((1,), jnp.float32),
        scratch_shapes=[pltpu.SMEM((2,), jnp.float32)],
    )(y_pred, y_true)
    return out[0]
